# baseline (speedup 1.0000x reference)
import jax
import jax.numpy as jnp
from jax.experimental import pallas as pl
from jax.experimental.pallas import tpu as pltpu


_NB = 1280
_NBLK = 16
_N = 20000


def _body(lg_ref, bx_ref, lab_ref, out_ref, prob_s, mx_s):
    p = pl.program_id(0)
    r = pl.program_id(1)
    C = lg_ref.shape[0]

    @pl.when(p == 0)
    def _phase_max():
        @pl.when(r == 0)
        def _init():
            mx_s[...] = jnp.zeros_like(mx_s)

        prob = jax.nn.sigmoid(lg_ref[...])
        prob_s[:, :, pl.ds(r * _NB, _NB)] = prob

        @pl.when(r < _NBLK - 1)
        def _full():
            mx_s[...] = jnp.maximum(mx_s[...], jnp.max(prob, axis=2))

        @pl.when(r == _NBLK - 1)
        def _edge():
            lane = jax.lax.broadcasted_iota(jnp.int32, prob.shape, 2)
            pm = jnp.where(lane < _N - r * _NB, prob, 0.0)
            mx_s[...] = jnp.maximum(mx_s[...], jnp.max(pm, axis=2))

    @pl.when(p == 1)
    def _phase_mask():
        prob = prob_s[:, :, pl.ds(r * _NB, _NB)]
        top = mx_s[...]
        labels = lab_ref[...]
        cls = jax.lax.broadcasted_iota(jnp.int32, (C,) + labels.shape, 0)
        present = jnp.any(labels[None] == cls, axis=2)
        thresh = jnp.where(present, 0.5 * top, 2.0)

        keep = prob >= thresh[:, :, None]
        scores = jnp.where(keep, prob, 0.0)
        box_keep = jnp.any(keep, axis=0)

        out_ref[:C] = scores
        bx = bx_ref[...]
        for coord in range(4):
            out_ref[C + coord] = jnp.where(box_keep, bx[:, coord, :], 0.0)


def kernel(pred_logits, pred_boxes, target_sizes, target_labels):
    del target_sizes
    B, N, C = pred_logits.shape
    lg = jnp.transpose(pred_logits, (2, 0, 1))
    bx = jnp.transpose(pred_boxes, (0, 2, 1))
    labels = target_labels.astype(jnp.int32)

    out_t = pl.pallas_call(
        _body,
        grid=(2, _NBLK),
        in_specs=[
            pl.BlockSpec((C, B, _NB), lambda p, r: (0, 0, r * (1 - p) + (_NBLK - 1) * p)),
            pl.BlockSpec((B, 4, _NB), lambda p, r: (0, 0, r * p)),
            pl.BlockSpec((B, 20), lambda p, r: (0, 0)),
        ],
        out_specs=pl.BlockSpec((C + 4, B, _NB), lambda p, r: (0, 0, r * p)),
        out_shape=jax.ShapeDtypeStruct((C + 4, B, N), jnp.float32),
        scratch_shapes=[
            pltpu.VMEM((C, B, _NB * _NBLK), jnp.float32),
            pltpu.VMEM((C, B), jnp.float32),
        ],
    )(lg, bx, labels)

    return jnp.transpose(out_t, (1, 2, 0))


# speedup vs baseline: 1.3535x; 1.3535x over previous
import jax
import jax.numpy as jnp
from jax.experimental import pallas as pl
from jax.experimental.pallas import tpu as pltpu


_NB = 4096
_NBLK = 5
_N = 20000


def _body(lg_ref, bx_ref, lab_ref, out_ref, prob_s, mx_s):
    p = pl.program_id(0)
    r = pl.program_id(1)
    C = lg_ref.shape[0]

    @pl.when(p == 0)
    def _phase_max():
        @pl.when(r == 0)
        def _init():
            mx_s[...] = jnp.zeros_like(mx_s)

        prob = jax.nn.sigmoid(lg_ref[...])
        prob_s[:, :, pl.ds(r * _NB, _NB)] = prob

        @pl.when(r < _NBLK - 1)
        def _full():
            mx_s[...] = jnp.maximum(mx_s[...], jnp.max(prob, axis=2))

        @pl.when(r == _NBLK - 1)
        def _edge():
            lane = jax.lax.broadcasted_iota(jnp.int32, prob.shape, 2)
            pm = jnp.where(lane < _N - r * _NB, prob, 0.0)
            mx_s[...] = jnp.maximum(mx_s[...], jnp.max(pm, axis=2))

    @pl.when(p == 1)
    def _phase_mask():
        prob = prob_s[:, :, pl.ds(r * _NB, _NB)]
        top = mx_s[...]
        labels = lab_ref[...]
        cls = jax.lax.broadcasted_iota(jnp.int32, (C,) + labels.shape, 0)
        present = jnp.any(labels[None] == cls, axis=2)
        thresh = jnp.where(present, 0.5 * top, 2.0)

        keep = prob >= thresh[:, :, None]
        scores = jnp.where(keep, prob, 0.0)
        box_keep = jnp.any(keep, axis=0)

        out_ref[:C] = scores
        bx = bx_ref[...]
        for coord in range(4):
            out_ref[C + coord] = jnp.where(box_keep, bx[:, coord, :], 0.0)


def kernel(pred_logits, pred_boxes, target_sizes, target_labels):
    del target_sizes
    B, N, C = pred_logits.shape
    lg = jnp.transpose(pred_logits, (2, 0, 1))
    bx = jnp.transpose(pred_boxes, (0, 2, 1))
    labels = target_labels.astype(jnp.int32)

    out_t = pl.pallas_call(
        _body,
        grid=(2, _NBLK),
        in_specs=[
            pl.BlockSpec((C, B, _NB), lambda p, r: (0, 0, r * (1 - p) + (_NBLK - 1) * p)),
            pl.BlockSpec((B, 4, _NB), lambda p, r: (0, 0, r * p)),
            pl.BlockSpec((B, 20), lambda p, r: (0, 0)),
        ],
        out_specs=pl.BlockSpec((C + 4, B, _NB), lambda p, r: (0, 0, r * p)),
        out_shape=jax.ShapeDtypeStruct((C + 4, B, N), jnp.float32),
        scratch_shapes=[
            pltpu.VMEM((C, B, _NB * _NBLK), jnp.float32),
            pltpu.VMEM((C, B), jnp.float32),
        ],
    )(lg, bx, labels)

    return jnp.transpose(out_t, (1, 2, 0))
